# Initial kernel scaffold; baseline (speedup 1.0000x reference)
import jax
import jax.numpy as jnp
from jax.experimental import pallas as pl


def kernel(y, x, f_y, W1, b1, W2, b2, W3, b3):
    raise NotImplementedError("write your pallas kernel here")


# speedup vs baseline: 3.2250x; 3.2250x over previous
import functools

import jax
import jax.numpy as jnp
from jax import lax
from jax.experimental import pallas as pl
from jax.experimental.pallas import tpu as pltpu
from jax.experimental.pallas import tpu_sc as plsc

RADIUS = 0.07
K = 12
NUM_FREQ = 32
MAX_POS = 10000.0
NPAD = 10240
BP = 1024
BM = 64
BQ = 256
NW = 32
EPW = NPAD * K // NW
CHUNK = 128


def _embed(c):
    kf = lax.broadcasted_iota(jnp.int32, (1, NUM_FREQ), 1).astype(jnp.float32)
    freqs = 1.0 / (MAX_POS ** (kf / NUM_FREQ))
    parts = []
    for d in range(3):
        ang = c[:, d:d + 1] * freqs
        parts.append(jnp.sin(ang))
        parts.append(jnp.cos(ang))
    return jnp.concatenate(parts, axis=1)


def _prep_body(c_ref, w_ref, b_ref, o_ref):
    emb = _embed(c_ref[...])
    o_ref[...] = (
        jnp.dot(emb, w_ref[...], preferred_element_type=jnp.float32) + b_ref[...]
    )


def _prep(coords, w, b):
    return pl.pallas_call(
        _prep_body,
        grid=(NPAD // BP,),
        in_specs=[
            pl.BlockSpec((BP, 8), lambda i: (i, 0)),
            pl.BlockSpec((192, 128), lambda i: (0, 0)),
            pl.BlockSpec((1, 128), lambda i: (0, 0)),
        ],
        out_specs=pl.BlockSpec((BP, 128), lambda i: (i, 0)),
        out_shape=jax.ShapeDtypeStruct((NPAD, 128), jnp.float32),
    )(coords, w, b)


def _search_body(x_ref, yt_ref, idx_ref, mask_ref):
    xb = x_ref[...]
    yt = yt_ref[...]
    xsq = (xb[:, 0:1] * xb[:, 0:1] + xb[:, 1:2] * xb[:, 1:2]) + xb[:, 2:3] * xb[:, 2:3]
    ysq = (yt[0:1, :] * yt[0:1, :] + yt[1:2, :] * yt[1:2, :]) + yt[2:3, :] * yt[2:3, :]
    prod = jnp.dot(xb, yt, preferred_element_type=jnp.float32)
    d2 = (xsq + ysq) - 2.0 * prod
    d2 = jnp.maximum(d2, 0.0)
    neg = jnp.where(d2 <= RADIUS * RADIUS, -d2, -jnp.inf)
    iota = lax.broadcasted_iota(jnp.int32, (BM, NPAD), 1)
    idx_ref[...] = jnp.zeros((BM, 128), jnp.int32)
    mask_ref[...] = jnp.zeros((BM, 128), jnp.float32)
    for kk in range(K):
        cm = jnp.max(neg, axis=1, keepdims=True)
        tie = neg == cm
        cidx = jnp.min(jnp.where(tie, iota, NPAD), axis=1, keepdims=True)
        valid = cm > -jnp.inf
        idx_ref[:, kk:kk + 1] = jnp.where(valid, cidx, 0)
        mask_ref[:, kk:kk + 1] = jnp.where(valid, 1.0, 0.0)
        neg = jnp.where(iota == cidx, -jnp.inf, neg)


def _search(xp, yt):
    return pl.pallas_call(
        _search_body,
        grid=(NPAD // BM,),
        in_specs=[
            pl.BlockSpec((BM, 8), lambda i: (i, 0)),
            pl.BlockSpec((8, NPAD), lambda i: (0, 0)),
        ],
        out_specs=[
            pl.BlockSpec((BM, 128), lambda i: (i, 0)),
            pl.BlockSpec((BM, 128), lambda i: (i, 0)),
        ],
        out_shape=[
            jax.ShapeDtypeStruct((NPAD, 128), jnp.int32),
            jax.ShapeDtypeStruct((NPAD, 128), jnp.float32),
        ],
    )(xp, yt)


def _make_gather():
    mesh = plsc.VectorSubcoreMesh(core_axis_name="c", subcore_axis_name="s")

    @functools.partial(
        pl.kernel,
        mesh=mesh,
        out_type=(
            jax.ShapeDtypeStruct((NPAD * K, 128), jnp.float32),
            jax.ShapeDtypeStruct((NPAD * K, 128), jnp.float32),
        ),
        scratch_types=[
            pltpu.VMEM((EPW,), jnp.int32),
            pltpu.VMEM((CHUNK, 128), jnp.float32),
            pltpu.VMEM((CHUNK, 128), jnp.float32),
            pltpu.SemaphoreType.DMA,
            pltpu.SemaphoreType.DMA,
        ],
    )
    def gather(a1_hbm, fy_hbm, idx_hbm, g1_hbm, gf_hbm, idx_v, bufa, buff, sema, semf):
        wid = lax.axis_index("s") * 2 + lax.axis_index("c")
        base = wid * EPW
        pltpu.sync_copy(idx_hbm.at[pl.ds(base, EPW)], idx_v)

        def body(c, carry):
            off = c * CHUNK
            isl = idx_v.at[pl.ds(off, CHUNK)]
            ca = pltpu.async_copy(a1_hbm.at[isl], bufa, sema)
            cf = pltpu.async_copy(fy_hbm.at[isl], buff, semf)
            ca.wait()
            cf.wait()
            pltpu.sync_copy(bufa, g1_hbm.at[pl.ds(base + off, CHUNK)])
            pltpu.sync_copy(buff, gf_hbm.at[pl.ds(base + off, CHUNK)])
            return carry

        lax.fori_loop(0, EPW // CHUNK, body, 0)

    return gather


def _mlp_body(g1_ref, gf_ref, a2_ref, mk_ref, w2_ref, b2_ref, w3_ref, b3_ref, o_ref):
    a2 = a2_ref[...]
    w2 = w2_ref[...]
    b2 = b2_ref[...]
    w3 = w3_ref[...]
    b3 = b3_ref[...]
    mk = mk_ref[...]
    acc = jnp.zeros((BQ, 128), jnp.float32)
    for kk in range(K):
        h = jax.nn.gelu(g1_ref[kk] + a2)
        h = jax.nn.gelu(jnp.dot(h, w2, preferred_element_type=jnp.float32) + b2)
        kv = (jnp.dot(h, w3, preferred_element_type=jnp.float32) + b3) * gf_ref[kk]
        acc = acc + kv * mk[:, kk:kk + 1]
    o_ref[...] = acc


def _mlp(g1, gf, a2, mk, w2, b2, w3, b3):
    return pl.pallas_call(
        _mlp_body,
        grid=(NPAD // BQ,),
        in_specs=[
            pl.BlockSpec((K, BQ, 128), lambda i: (0, i, 0)),
            pl.BlockSpec((K, BQ, 128), lambda i: (0, i, 0)),
            pl.BlockSpec((BQ, 128), lambda i: (i, 0)),
            pl.BlockSpec((BQ, 128), lambda i: (i, 0)),
            pl.BlockSpec((128, 256), lambda i: (0, 0)),
            pl.BlockSpec((1, 256), lambda i: (0, 0)),
            pl.BlockSpec((256, 128), lambda i: (0, 0)),
            pl.BlockSpec((1, 128), lambda i: (0, 0)),
        ],
        out_specs=pl.BlockSpec((BQ, 128), lambda i: (i, 0)),
        out_shape=jax.ShapeDtypeStruct((NPAD, 128), jnp.float32),
    )(g1, gf, a2, mk, w2, b2, w3, b3)


def kernel(y, x, f_y, W1, b1, W2, b2, W3, b3):
    n = y.shape[0]
    m = x.shape[0]
    yp = jnp.pad(jnp.pad(y, ((0, 0), (0, 5))), ((0, NPAD - n), (0, 0)),
                 constant_values=100.0)
    xp = jnp.pad(jnp.pad(x, ((0, 0), (0, 5))), ((0, NPAD - m), (0, 0)))
    fyp = jnp.pad(f_y, ((0, NPAD - n), (0, 0)))

    A1 = _prep(yp, W1[:192], jnp.zeros((1, 128), jnp.float32))
    A2 = _prep(xp, W1[192:], b1[None, :])
    idxw, maskw = _search(xp, yp.T)
    flat_idx = idxw[:, :K].T.reshape(-1)
    g1, gf = _make_gather()(A1, fyp, flat_idx)
    g1 = g1.reshape(K, NPAD, 128)
    gf = gf.reshape(K, NPAD, 128)
    out = _mlp(g1, gf, A2, maskw, W2, b2[None, :], W3, b3[None, :])
    return out[:m]
